# baseline (speedup 1.0000x reference)
import jax
import jax.numpy as jnp
from jax.experimental import pallas as pl
from jax.experimental.pallas import tpu as pltpu


_FC = 1024
_PREC = jax.lax.Precision.DEFAULT


def _body(xf_ref, wg_ref, w1_ref, b1_ref, w2_ref, b2_ref,
          xb_ref, we1_ref, be1_ref, we2_ref, be2_ref,
          o_ref, gall_ref):
    u = pl.program_id(0)
    f = pl.program_id(1)
    Tb, E = wg_ref.shape
    T = xf_ref.shape[0]
    split = T - Tb

    @pl.when(jnp.logical_and(u == 0, f == 0))
    def _():
        g = jax.nn.softmax(wg_ref[:], axis=-1)
        lane = jax.lax.broadcasted_iota(jnp.int32, g.shape, 1)
        for i in range(E):
            gall_ref[i] = jnp.sum(jnp.where(lane == i, g, 0.0), axis=1,
                                  keepdims=True)
        o_ref[:] = jnp.zeros_like(o_ref) + b2_ref[:]
        o_ref[split:, :] += jnp.dot(g, be2_ref[:], precision=_PREC,
                                    preferred_element_type=jnp.float32)

    @pl.when(u == 0)
    def _():
        h = jnp.dot(xf_ref[:], w1_ref[:], precision=_PREC,
                    preferred_element_type=jnp.float32)
        h = jax.nn.gelu((h + b1_ref[:]).astype(jnp.bfloat16))
        o_ref[:] += jnp.dot(h, w2_ref[:], precision=_PREC,
                            preferred_element_type=jnp.float32)

    @pl.when(u > 0)
    def _():
        ge = gall_ref[u - 1]
        h = jnp.dot(xb_ref[:], we1_ref[0], precision=_PREC,
                    preferred_element_type=jnp.float32)
        h = jax.nn.gelu((h + be1_ref[0]).astype(jnp.bfloat16))
        o_ref[split:, :] += ge * jnp.dot(h, we2_ref[0], precision=_PREC,
                                         preferred_element_type=jnp.float32)


def kernel(x, id, weight, W1, b1, W2, b2, We1, be1, We2, be2):
    B, T, D = x.shape
    Tb, E = weight.shape[1], weight.shape[2]
    F = W1.shape[1]
    fc = _FC
    nf = F // fc
    x2 = x.reshape(T, D)
    wg2 = weight.reshape(Tb, E)

    out = pl.pallas_call(
        _body,
        grid=(E + 1, nf),
        in_specs=[
            pl.BlockSpec((T, D), lambda u, f: (0, 0)),
            pl.BlockSpec((Tb, E), lambda u, f: (0, 0)),
            pl.BlockSpec((D, fc),
                         lambda u, f: (0, jnp.where(u == 0, f, nf - 1))),
            pl.BlockSpec((1, fc),
                         lambda u, f: (0, jnp.where(u == 0, f, nf - 1))),
            pl.BlockSpec((fc, D),
                         lambda u, f: (jnp.where(u == 0, f, nf - 1), 0)),
            pl.BlockSpec((1, D), lambda u, f: (0, 0)),
            pl.BlockSpec((Tb, D), lambda u, f: (1, 0)),
            pl.BlockSpec((1, D, fc),
                         lambda u, f: (jnp.maximum(u - 1, 0), 0,
                                       jnp.where(u == 0, 0, f))),
            pl.BlockSpec((1, 1, fc),
                         lambda u, f: (jnp.maximum(u - 1, 0), 0,
                                       jnp.where(u == 0, 0, f))),
            pl.BlockSpec((1, fc, D),
                         lambda u, f: (jnp.maximum(u - 1, 0),
                                       jnp.where(u == 0, 0, f), 0)),
            pl.BlockSpec((E, D), lambda u, f: (0, 0)),
        ],
        out_specs=pl.BlockSpec((T, D), lambda u, f: (0, 0)),
        out_shape=jax.ShapeDtypeStruct((T, D), jnp.float32),
        scratch_shapes=[
            pltpu.VMEM((E, Tb, 1), jnp.float32),
        ],
        compiler_params=pltpu.CompilerParams(
            dimension_semantics=("arbitrary", "arbitrary")),
    )(x2, wg2, W1, b1.reshape(1, F), W2, b2.reshape(1, D),
      x2, We1, be1.reshape(E, 1, F), We2, be2)

    return out.reshape(B, T, D)
